# baseline (speedup 1.0000x reference)
import functools

import jax
import jax.numpy as jnp
from jax import lax
from jax.experimental import pallas as pl
from jax.experimental.pallas import tpu as pltpu
from jax.experimental.pallas import tpu_sc as plsc

_D = 64
_HIST = 50
_L = 16
_CB = 128
_NBUF = 4
_NB = 1024
_TB = 2048
_PB = 10000
_DP = 128


def _sc_geometry():
    try:
        info = plsc.get_sparse_core_info()
        return info.num_cores, info.num_subcores
    except Exception:
        return 2, 16


@functools.lru_cache(maxsize=None)
def _make_pooler(batch):
    nc, ns = _sc_geometry()
    nw = nc * ns
    bw = batch // nw
    nchunk_b = bw // _CB
    nchunk = _HIST * nchunk_b
    mesh = plsc.VectorSubcoreMesh(core_axis_name="c", subcore_axis_name="s")

    @functools.partial(
        pl.kernel,
        mesh=mesh,
        out_type=jax.ShapeDtypeStruct((batch, _DP), jnp.float32),
        scratch_types=[
            pltpu.VMEM((_HIST, bw), jnp.int32),
            pltpu.VMEM((bw, _DP), jnp.float32),
            *[pltpu.SemaphoreType.DMA for _ in range(_NBUF)],
        ],
        compiler_params=pltpu.CompilerParams(use_tc_tiling_on_sc=True),
    )
    def pool(idx_hbm, table_hbm, out_hbm, idx_v, acc_v, *sems):
        w = lax.axis_index("s") * nc + lax.axis_index("c")
        base = w * bw
        pltpu.sync_copy(idx_hbm.at[:, pl.ds(base, bw)], idx_v)

        def idx_ref(c):
            j = c // nchunk_b
            b0 = (c % nchunk_b) * _CB
            return idx_v.at[j, pl.ds(b0, _CB)]

        def acc_ref(c):
            b0 = (c % nchunk_b) * _CB
            return acc_v.at[pl.ds(b0, _CB), :]

        def start(c, b, add=True):
            pltpu.async_copy(table_hbm.at[idx_ref(c)], acc_ref(c), sems[b], add=add)

        def drain(c, b):
            pltpu.make_async_copy(table_hbm.at[idx_ref(c)], acc_ref(c), sems[b]).wait()

        assert _NBUF == nchunk_b
        for b in range(_NBUF):
            start(b, b, add=False)

        def group(gi, carry):
            c0 = gi * _NBUF
            for b in range(_NBUF):
                c = c0 + b
                drain(c, b)

                @pl.when(c + _NBUF < nchunk)
                def _():
                    start(c + _NBUF, b)

            return carry

        lax.fori_loop(0, nchunk // _NBUF, group, 0)
        pltpu.sync_copy(acc_v, out_hbm.at[pl.ds(base, bw), :])

    return pool, nw


def _pad_kernel(x_ref, o_ref):
    o_ref[:, : _D] = x_ref[...]
    o_ref[:, _D:] = jnp.zeros((x_ref.shape[0], _DP - _D), jnp.float32)


@functools.lru_cache(maxsize=None)
def _make_padder(n):
    pb = _PB
    while n % pb:
        pb //= 2
    return pl.pallas_call(
        _pad_kernel,
        grid=(n // pb,),
        in_specs=[pl.BlockSpec((pb, _D), lambda i: (i, 0))],
        out_specs=pl.BlockSpec((pb, _DP), lambda i: (i, 0)),
        out_shape=jax.ShapeDtypeStruct((n, _DP), jnp.float32),
    )


def _tr_kernel(x_ref, o_ref):
    o_ref[...] = x_ref[...].T


@functools.lru_cache(maxsize=None)
def _make_transposer(batch):
    return pl.pallas_call(
        _tr_kernel,
        grid=(batch // _TB,),
        in_specs=[pl.BlockSpec((_TB, _HIST), lambda i: (i, 0))],
        out_specs=pl.BlockSpec((_HIST, _TB), lambda i: (0, i)),
        out_shape=jax.ShapeDtypeStruct((_HIST, batch), jnp.int32),
    )


def _norm_kernel(x_ref, o_ref):
    x = x_ref[:, : _D]
    nsq = jnp.sum(x * x, axis=1, keepdims=True)
    clamp = jnp.float32((_HIST * 1e-12) ** 2)
    o_ref[...] = x * lax.rsqrt(jnp.maximum(nsq, clamp))


@functools.lru_cache(maxsize=None)
def _make_normalizer(batch):
    return pl.pallas_call(
        _norm_kernel,
        grid=(batch // _NB,),
        in_specs=[pl.BlockSpec((_NB, _DP), lambda i: (i, 0))],
        out_specs=pl.BlockSpec((_NB, _D), lambda i: (i, 0)),
        out_shape=jax.ShapeDtypeStruct((batch, _D), jnp.float32),
    )


def kernel(indices, embeddings):
    b, h = indices.shape
    assert h == _HIST and embeddings.shape[1] == _D
    pool, nw = _make_pooler(b)
    idx_t = _make_transposer(b)(indices.astype(jnp.int32))
    table = _make_padder(embeddings.shape[0])(embeddings.astype(jnp.float32))
    sums = pool(idx_t, table)
    return _make_normalizer(b)(sums)


# speedup vs baseline: 1.8838x; 1.8838x over previous
import functools

import jax
import jax.numpy as jnp
from jax import lax
from jax.experimental import pallas as pl
from jax.experimental.pallas import tpu as pltpu
from jax.experimental.pallas import tpu_sc as plsc

_D = 64
_HIST = 50
_L = 16
_CB = 128
_NBUF = 4
_NB = 1024
_TB = 2048
_PB = 8192
_DP = 128


def _sc_geometry():
    try:
        info = plsc.get_sparse_core_info()
        return info.num_cores, info.num_subcores
    except Exception:
        return 2, 16


@functools.lru_cache(maxsize=None)
def _make_pooler(batch):
    nc, ns = _sc_geometry()
    nw = nc * ns
    bw = batch // nw
    nchunk_b = bw // _CB
    nchunk = _HIST * nchunk_b
    mesh = plsc.VectorSubcoreMesh(core_axis_name="c", subcore_axis_name="s")

    @functools.partial(
        pl.kernel,
        mesh=mesh,
        out_type=jax.ShapeDtypeStruct((batch, _DP), jnp.float32),
        scratch_types=[
            pltpu.VMEM((_HIST, bw), jnp.int32),
            pltpu.VMEM((bw, _DP), jnp.float32),
            *[pltpu.SemaphoreType.DMA for _ in range(_NBUF)],
        ],
        compiler_params=pltpu.CompilerParams(use_tc_tiling_on_sc=True),
    )
    def pool(idx_hbm, table_hbm, out_hbm, idx_v, acc_v, *sems):
        w = lax.axis_index("s") * nc + lax.axis_index("c")
        base = w * bw
        pltpu.sync_copy(idx_hbm.at[:, pl.ds(base, bw)], idx_v)

        def idx_ref(c):
            j = c // nchunk_b
            b0 = (c % nchunk_b) * _CB
            return idx_v.at[j, pl.ds(b0, _CB)]

        def acc_ref(c):
            b0 = (c % nchunk_b) * _CB
            return acc_v.at[pl.ds(b0, _CB), :]

        def start(c, b, add=True):
            pltpu.async_copy(table_hbm.at[idx_ref(c)], acc_ref(c), sems[b], add=add)

        def drain(c, b):
            pltpu.make_async_copy(table_hbm.at[idx_ref(c)], acc_ref(c), sems[b]).wait()

        assert _NBUF == nchunk_b
        for b in range(_NBUF):
            start(b, b, add=False)

        def group(gi, carry):
            c0 = gi * _NBUF
            for b in range(_NBUF):
                c = c0 + b
                drain(c, b)

                @pl.when(c + _NBUF < nchunk)
                def _():
                    start(c + _NBUF, b)

            return carry

        lax.fori_loop(0, nchunk // _NBUF, group, 0)
        pltpu.sync_copy(acc_v, out_hbm.at[pl.ds(base, bw), :])

    return pool, nw


def _padtr_kernel(x_ref, o_ref):
    o_ref[:, : _D] = x_ref[...].T
    o_ref[:, _D:] = jnp.zeros((o_ref.shape[0], _DP - _D), jnp.float32)


@functools.lru_cache(maxsize=None)
def _make_padtr(n):
    pb = _PB
    return pl.pallas_call(
        _padtr_kernel,
        grid=((n + pb - 1) // pb,),
        in_specs=[pl.BlockSpec((_D, pb), lambda i: (0, i))],
        out_specs=pl.BlockSpec((pb, _DP), lambda i: (i, 0)),
        out_shape=jax.ShapeDtypeStruct((n, _DP), jnp.float32),
    )


def _norm_kernel(x_ref, o_ref):
    x = x_ref[:, : _D]
    nsq = jnp.sum(x * x, axis=1, keepdims=True)
    clamp = jnp.float32((_HIST * 1e-12) ** 2)
    o_ref[...] = x * lax.rsqrt(jnp.maximum(nsq, clamp))


@functools.lru_cache(maxsize=None)
def _make_normalizer(batch):
    return pl.pallas_call(
        _norm_kernel,
        grid=(batch // _NB,),
        in_specs=[pl.BlockSpec((_NB, _DP), lambda i: (i, 0))],
        out_specs=pl.BlockSpec((_NB, _D), lambda i: (i, 0)),
        out_shape=jax.ShapeDtypeStruct((batch, _D), jnp.float32),
    )


def kernel(indices, embeddings):
    b, h = indices.shape
    assert h == _HIST and embeddings.shape[1] == _D
    pool, nw = _make_pooler(b)
    idx_t = indices.astype(jnp.int32).T
    table = _make_padtr(embeddings.shape[0])(embeddings.astype(jnp.float32).T)
    sums = pool(idx_t, table)
    return _make_normalizer(b)(sums)


# speedup vs baseline: 1.9147x; 1.0164x over previous
import functools

import jax
import jax.numpy as jnp
from jax import lax
from jax.experimental import pallas as pl
from jax.experimental.pallas import tpu as pltpu
from jax.experimental.pallas import tpu_sc as plsc

_D = 64
_HIST = 50
_L = 16
_CB = 128
_NBUF = 4
_NB = 1024
_TB = 2048
_PB = 8192
_DP = 128


def _sc_geometry():
    try:
        info = plsc.get_sparse_core_info()
        return info.num_cores, info.num_subcores
    except Exception:
        return 2, 16


@functools.lru_cache(maxsize=None)
def _make_pooler(batch):
    nc, ns = _sc_geometry()
    nw = nc * ns
    bw = batch // nw
    nchunk_b = bw // _CB
    nchunk = _HIST * nchunk_b
    mesh = plsc.VectorSubcoreMesh(core_axis_name="c", subcore_axis_name="s")

    @functools.partial(
        pl.kernel,
        mesh=mesh,
        out_type=jax.ShapeDtypeStruct((batch, _DP), jnp.float32),
        scratch_types=[
            pltpu.VMEM((_HIST, bw), jnp.int32),
            pltpu.VMEM((bw, _DP), jnp.float32),
            *[pltpu.SemaphoreType.DMA for _ in range(_NBUF)],
        ],
        compiler_params=pltpu.CompilerParams(use_tc_tiling_on_sc=True),
    )
    def pool(idx_hbm, table_hbm, out_hbm, idx_v, acc_v, *sems):
        w = lax.axis_index("s") * nc + lax.axis_index("c")
        base = w * bw
        pltpu.sync_copy(idx_hbm.at[:, pl.ds(base, bw)], idx_v)

        def idx_ref(c):
            j = c // nchunk_b
            b0 = (c % nchunk_b) * _CB
            return idx_v.at[j, pl.ds(b0, _CB)]

        def acc_ref(c):
            b0 = (c % nchunk_b) * _CB
            return acc_v.at[pl.ds(b0, _CB), :]

        def start(c, b, add=True):
            pltpu.async_copy(table_hbm.at[idx_ref(c)], acc_ref(c), sems[b], add=add)

        def drain(c, b):
            pltpu.make_async_copy(table_hbm.at[idx_ref(c)], acc_ref(c), sems[b]).wait()

        assert _NBUF == nchunk_b
        for b in range(_NBUF):
            start(b, b, add=False)

        def group(gi, carry):
            c0 = gi * _NBUF
            for b in range(_NBUF):
                c = c0 + b
                drain(c, b)

                @pl.when(c + _NBUF < nchunk)
                def _():
                    start(c + _NBUF, b)

            return carry

        lax.fori_loop(0, nchunk // _NBUF, group, 0)
        pltpu.sync_copy(acc_v, out_hbm.at[pl.ds(base, bw), :])

    return pool, nw


def _padtr_kernel(x_ref, o_ref):
    o_ref[:, : _D] = x_ref[...].T


@functools.lru_cache(maxsize=None)
def _make_padtr(n):
    pb = _PB
    return pl.pallas_call(
        _padtr_kernel,
        grid=((n + pb - 1) // pb,),
        in_specs=[pl.BlockSpec((_D, pb), lambda i: (0, i))],
        out_specs=pl.BlockSpec((pb, _DP), lambda i: (i, 0)),
        out_shape=jax.ShapeDtypeStruct((n, _DP), jnp.float32),
    )


def _norm_kernel(x_ref, o_ref):
    x = x_ref[:, : _D]
    nsq = jnp.sum(x * x, axis=1, keepdims=True)
    clamp = jnp.float32((_HIST * 1e-12) ** 2)
    o_ref[...] = (x * lax.rsqrt(jnp.maximum(nsq, clamp))).T


@functools.lru_cache(maxsize=None)
def _make_normalizer(batch):
    return pl.pallas_call(
        _norm_kernel,
        grid=(batch // _NB,),
        in_specs=[pl.BlockSpec((_NB, _DP), lambda i: (i, 0))],
        out_specs=pl.BlockSpec((_D, _NB), lambda i: (0, i)),
        out_shape=jax.ShapeDtypeStruct((_D, batch), jnp.float32),
    )


def kernel(indices, embeddings):
    b, h = indices.shape
    assert h == _HIST and embeddings.shape[1] == _D
    pool, nw = _make_pooler(b)
    idx_t = indices.astype(jnp.int32).T
    table = _make_padtr(embeddings.shape[0])(embeddings.astype(jnp.float32).T)
    sums = pool(idx_t, table)
    return _make_normalizer(b)(sums).T


# speedup vs baseline: 2.0150x; 1.0524x over previous
import functools

import jax
import jax.numpy as jnp
from jax import lax
from jax.experimental import pallas as pl
from jax.experimental.pallas import tpu as pltpu
from jax.experimental.pallas import tpu_sc as plsc

_D = 64
_HIST = 50
_L = 16
_CB = 128
_NBUF = 4
_NB = 1024
_TB = 2048
_PB = 16384
_DP = 128


def _sc_geometry():
    try:
        info = plsc.get_sparse_core_info()
        return info.num_cores, info.num_subcores
    except Exception:
        return 2, 16


@functools.lru_cache(maxsize=None)
def _make_pooler(batch):
    nc, ns = _sc_geometry()
    nw = nc * ns
    bw = batch // nw
    nchunk_b = bw // _CB
    nchunk = _HIST * nchunk_b
    mesh = plsc.VectorSubcoreMesh(core_axis_name="c", subcore_axis_name="s")

    @functools.partial(
        pl.kernel,
        mesh=mesh,
        out_type=jax.ShapeDtypeStruct((batch, _DP), jnp.float32),
        scratch_types=[
            pltpu.VMEM((_HIST, bw), jnp.int32),
            pltpu.VMEM((bw, _DP), jnp.float32),
            *[pltpu.SemaphoreType.DMA for _ in range(_NBUF)],
        ],
        compiler_params=pltpu.CompilerParams(use_tc_tiling_on_sc=True),
    )
    def pool(idx_hbm, table_hbm, out_hbm, idx_v, acc_v, *sems):
        w = lax.axis_index("s") * nc + lax.axis_index("c")
        base = w * bw
        pltpu.sync_copy(idx_hbm.at[:, pl.ds(base, bw)], idx_v)

        def idx_ref(c):
            j = c // nchunk_b
            b0 = (c % nchunk_b) * _CB
            return idx_v.at[j, pl.ds(b0, _CB)]

        def acc_ref(c):
            b0 = (c % nchunk_b) * _CB
            return acc_v.at[pl.ds(b0, _CB), :]

        def start(c, b, add=True):
            pltpu.async_copy(table_hbm.at[idx_ref(c)], acc_ref(c), sems[b], add=add)

        def drain(c, b):
            pltpu.make_async_copy(table_hbm.at[idx_ref(c)], acc_ref(c), sems[b]).wait()

        assert _NBUF == nchunk_b
        for b in range(_NBUF):
            start(b, b, add=False)

        def group(gi, carry):
            c0 = gi * _NBUF
            for b in range(_NBUF):
                c = c0 + b
                drain(c, b)

                @pl.when(c + _NBUF < nchunk)
                def _():
                    start(c + _NBUF, b)

            return carry

        lax.fori_loop(0, nchunk // _NBUF, group, 0)
        pltpu.sync_copy(acc_v, out_hbm.at[pl.ds(base, bw), :])

    return pool, nw


def _padtr_kernel(x_ref, o_ref):
    o_ref[:, : _D] = x_ref[...].T


@functools.lru_cache(maxsize=None)
def _make_padtr(n):
    pb = _PB
    return pl.pallas_call(
        _padtr_kernel,
        grid=((n + pb - 1) // pb,),
        in_specs=[pl.BlockSpec((_D, pb), lambda i: (0, i))],
        out_specs=pl.BlockSpec((pb, _DP), lambda i: (i, 0)),
        out_shape=jax.ShapeDtypeStruct((n, _DP), jnp.float32),
    )


def _norm_kernel(x_ref, o_ref):
    x = x_ref[:, : _D]
    nsq = jnp.sum(x * x, axis=1, keepdims=True)
    clamp = jnp.float32((_HIST * 1e-12) ** 2)
    o_ref[...] = (x * lax.rsqrt(jnp.maximum(nsq, clamp))).T


@functools.lru_cache(maxsize=None)
def _make_normalizer(batch):
    return pl.pallas_call(
        _norm_kernel,
        grid=(batch // _NB,),
        in_specs=[pl.BlockSpec((_NB, _DP), lambda i: (i, 0))],
        out_specs=pl.BlockSpec((_D, _NB), lambda i: (0, i)),
        out_shape=jax.ShapeDtypeStruct((_D, batch), jnp.float32),
    )


def kernel(indices, embeddings):
    b, h = indices.shape
    assert h == _HIST and embeddings.shape[1] == _D
    pool, nw = _make_pooler(b)
    idx_t = indices.astype(jnp.int32).T
    table = _make_padtr(embeddings.shape[0])(embeddings.astype(jnp.float32).T)
    sums = pool(idx_t, table)
    return _make_normalizer(b)(sums).T
